# baseline (speedup 1.0000x reference)
import jax
import jax.numpy as jnp
from jax import lax
from jax.experimental import pallas as pl
from jax.experimental.pallas import tpu as pltpu
from jax.experimental.pallas import tpu_sc as plsc

EMB_K = 16
BATCH = 16384

_NC = 2
_NS = 16
_NW = _NC * _NS
_BPW = BATCH // _NW
_CHUNK = 128
_NCHUNK = _BPW // _CHUNK


def _body(x_ref, w_ref, h_ref, out_ref, xv, uidx, vidx, urows, vrows, prod_t,
          outv, sem):
    wid = lax.axis_index("s") * _NC + lax.axis_index("c")
    base = wid * _BPW

    pltpu.sync_copy(x_ref.at[pl.ds(base * 2, _BPW * 2)], xv)

    iota = lax.iota(jnp.int32, 16)

    for j in range(_BPW // 16):
        pairs = j * 32 + iota * 2
        uidx[pl.ds(j * 16, 16)] = plsc.load_gather(xv, [pairs])
        vidx[pl.ds(j * 16, 16)] = plsc.load_gather(xv, [pairs + 1])

    copies = []
    for c in range(_NCHUNK):
        sl = pl.ds(c * _CHUNK, _CHUNK)
        copies.append(pltpu.async_copy(w_ref.at[uidx.at[sl]], urows.at[sl], sem))
        copies.append(pltpu.async_copy(h_ref.at[vidx.at[sl]], vrows.at[sl], sem))
    for cp in copies:
        cp.wait()

    for i in range(_BPW):
        p = urows[i] * vrows[i]
        plsc.store_scatter(prod_t, [iota * _BPW + i], p)

    one = jnp.full((16,), 1.0, jnp.float32)

    for b in range(_BPW // 16):
        acc = prod_t[pl.ds(b * 16, 16)]
        for k in range(1, EMB_K):
            acc = acc + prod_t[pl.ds(k * _BPW + b * 16, 16)]
        outv[pl.ds(b * 16, 16)] = one / (one + jnp.exp(-acc))

    pltpu.sync_copy(outv, out_ref.at[pl.ds(base, _BPW)])


@jax.jit
def _mf_sc(x, W, H):
    mesh = plsc.VectorSubcoreMesh(core_axis_name="c", subcore_axis_name="s")
    return pl.kernel(
        _body,
        mesh=mesh,
        compiler_params=pltpu.CompilerParams(needs_layout_passes=False,
                                              use_tc_tiling_on_sc=False),
        out_type=jax.ShapeDtypeStruct((BATCH,), jnp.float32),
        scratch_types=[
            pltpu.VMEM((_BPW * 2,), jnp.int32),
            pltpu.VMEM((_BPW,), jnp.int32),
            pltpu.VMEM((_BPW,), jnp.int32),
            pltpu.VMEM((_BPW, EMB_K), jnp.float32),
            pltpu.VMEM((_BPW, EMB_K), jnp.float32),
            pltpu.VMEM((_BPW * EMB_K,), jnp.float32),
            pltpu.VMEM((_BPW,), jnp.float32),
            pltpu.SemaphoreType.DMA,
        ],
    )(x.reshape(-1), W, H)


def kernel(x, W, H):
    return _mf_sc(x, W, H)
